# baseline (device time: 205741 ns/iter reference)
import jax
import jax.numpy as jnp
from jax import lax
from jax.experimental import pallas as pl
from jax.experimental.pallas import tpu as pltpu

N_DEV = 4
T = 4096
D = 1024
MR = 1280
KB = 512


def _a2a_body(x_ref, q_ref, cnt_ref, recv_ref, cntrecv_ref,
              chunks_ref, dsend, drecv, csend, crecv):
    me = lax.axis_index("i")

    barrier = pltpu.get_barrier_semaphore()
    for k in (1, 2, 3):
        pl.semaphore_signal(
            barrier, inc=1,
            device_id=((me + k) % N_DEV,),
            device_id_type=pl.DeviceIdType.MESH,
        )
    pl.semaphore_wait(barrier, 3)

    cntrecv_ref[0, :] = cnt_ref[0, :]
    crdmas = []
    for k in (1, 2, 3):
        dst = (me + k) % N_DEV
        c = pltpu.make_async_remote_copy(
            src_ref=cnt_ref.at[0],
            dst_ref=cntrecv_ref.at[k],
            send_sem=csend.at[k],
            recv_sem=crecv.at[k],
            device_id=(dst,),
            device_id_type=pl.DeviceIdType.MESH,
        )
        c.start()
        crdmas.append(c)

    def build_chunk(d):
        def inner(j, acc):
            qb = q_ref[j]
            rows = jax.lax.broadcasted_iota(jnp.int32, (MR, KB), 0) + d * MR
            s = (rows == qb).astype(jnp.bfloat16)
            xb = x_ref[pl.ds(j * KB, KB), :]
            return acc + jax.lax.dot_general(
                s, xb, (((1,), (0,)), ((), ())),
                preferred_element_type=jnp.float32,
            )
        acc = lax.fori_loop(0, T // KB, inner,
                            jnp.zeros((MR, D), jnp.float32))
        return acc.astype(jnp.bfloat16)

    drdmas = []
    for k in (1, 2, 3):
        dst = (me + k) % N_DEV
        chunks_ref[k, :, :] = build_chunk(dst)
        d = pltpu.make_async_remote_copy(
            src_ref=chunks_ref.at[k],
            dst_ref=recv_ref.at[k],
            send_sem=dsend.at[k],
            recv_sem=drecv.at[k],
            device_id=(dst,),
            device_id_type=pl.DeviceIdType.MESH,
        )
        d.start()
        drdmas.append(d)

    recv_ref[0, :, :] = build_chunk(me)

    for c in crdmas:
        c.wait_send()
        c.wait_recv()
    for d in drdmas:
        d.wait()


def _compact_body(recv_ref, idx_ref, out_ref):
    def outer(rb, _):
        base = rb * KB
        idxb = idx_ref[pl.ds(base, KB), :]
        def inner(j, acc):
            cols = jax.lax.broadcasted_iota(jnp.int32, (KB, KB), 1) + j * KB
            g = (cols == idxb).astype(jnp.bfloat16)
            rblk = recv_ref[pl.ds(j * KB, KB), :]
            return acc + jax.lax.dot_general(
                g, rblk, (((1,), (0,)), ((), ())),
                preferred_element_type=jnp.float32,
            )
        acc = lax.fori_loop(0, N_DEV * MR // KB, inner,
                            jnp.zeros((KB, D), jnp.float32))
        out_ref[pl.ds(base, KB), :] = acc.astype(jnp.bfloat16)
        return 0
    lax.fori_loop(0, T // KB, outer, 0)


def kernel(x, dest):
    onehot = (dest[:, None] == jnp.arange(N_DEV)[None, :]).astype(jnp.int32)
    counts = onehot.sum(axis=0).astype(jnp.int32)
    ranks = (jnp.cumsum(onehot, axis=0) * onehot).sum(axis=1) - 1
    q = (dest.astype(jnp.int32) * MR + ranks).astype(jnp.int32)
    x_bf = x.astype(jnp.bfloat16)
    cnt_payload = jnp.zeros((1, 128), jnp.int32).at[0, :N_DEV].set(counts)

    recv, cnt_recv = pl.pallas_call(
        _a2a_body,
        out_shape=[
            jax.ShapeDtypeStruct((N_DEV, MR, D), jnp.bfloat16),
            jax.ShapeDtypeStruct((N_DEV, 128), jnp.int32),
        ],
        in_specs=[
            pl.BlockSpec(memory_space=pltpu.VMEM),
            pl.BlockSpec(memory_space=pltpu.VMEM),
            pl.BlockSpec(memory_space=pltpu.VMEM),
        ],
        out_specs=[
            pl.BlockSpec(memory_space=pltpu.VMEM),
            pl.BlockSpec(memory_space=pltpu.VMEM),
        ],
        scratch_shapes=[
            pltpu.VMEM((N_DEV, MR, D), jnp.bfloat16),
            pltpu.SemaphoreType.DMA((N_DEV,)),
            pltpu.SemaphoreType.DMA((N_DEV,)),
            pltpu.SemaphoreType.DMA((N_DEV,)),
            pltpu.SemaphoreType.DMA((N_DEV,)),
        ],
        compiler_params=pltpu.CompilerParams(collective_id=0),
    )(x_bf, q.reshape(T // KB, 1, KB), cnt_payload)

    me = lax.axis_index("i")
    c_slot = jnp.take(cnt_recv[:, :N_DEV], me, axis=1)
    slot_of_src = (me - jnp.arange(N_DEV)) % N_DEV
    c_src = c_slot[slot_of_src]
    cum = jnp.cumsum(c_src)
    starts = cum - c_src
    r = jnp.arange(T)
    p_of_r = (r[:, None] >= cum[None, :]).astype(jnp.int32).sum(axis=1)
    idx = slot_of_src[p_of_r] * MR + (r - starts[p_of_r])

    out = pl.pallas_call(
        _compact_body,
        out_shape=jax.ShapeDtypeStruct((T, D), jnp.bfloat16),
        in_specs=[
            pl.BlockSpec(memory_space=pltpu.VMEM),
            pl.BlockSpec(memory_space=pltpu.VMEM),
        ],
        out_specs=pl.BlockSpec(memory_space=pltpu.VMEM),
    )(recv.reshape(N_DEV * MR, D), idx.astype(jnp.int32).reshape(T, 1))
    return out


# device time: 127502 ns/iter; 1.6136x vs baseline; 1.6136x over previous
import jax
import jax.numpy as jnp
from jax import lax
from jax.experimental import pallas as pl
from jax.experimental.pallas import tpu as pltpu

N_DEV = 4
T = 4096
D = 1024
MR = 1152
KB1 = 1024
KB = 512
NJ = N_DEV * MR // KB
NR = T // KB


def _a2a_body(x_ref, q_ref, cnt_ref, recv_ref, cntrecv_ref,
              chunks_ref, dsend, drecv, csend, crecv):
    me = lax.axis_index("i")

    barrier = pltpu.get_barrier_semaphore()
    for k in (1, 2, 3):
        pl.semaphore_signal(
            barrier, inc=1,
            device_id=((me + k) % N_DEV,),
            device_id_type=pl.DeviceIdType.MESH,
        )
    pl.semaphore_wait(barrier, 3)

    cntrecv_ref[0, :] = cnt_ref[0, :]
    crdmas = []
    for k in (1, 2, 3):
        dst = (me + k) % N_DEV
        c = pltpu.make_async_remote_copy(
            src_ref=cnt_ref.at[0],
            dst_ref=cntrecv_ref.at[k],
            send_sem=csend.at[k],
            recv_sem=crecv.at[k],
            device_id=(dst,),
            device_id_type=pl.DeviceIdType.MESH,
        )
        c.start()
        crdmas.append(c)

    def build_chunk(d):
        def inner(j, acc):
            qb = q_ref[j]
            rows = jax.lax.broadcasted_iota(jnp.int32, (MR, KB1), 0) + d * MR
            s = (rows == qb).astype(jnp.bfloat16)
            xb = x_ref[pl.ds(j * KB1, KB1), :]
            return acc + jax.lax.dot_general(
                s, xb, (((1,), (0,)), ((), ())),
                preferred_element_type=jnp.float32,
            ).astype(jnp.bfloat16)
        return lax.fori_loop(0, T // KB1, inner,
                             jnp.zeros((MR, D), jnp.bfloat16))

    drdmas = []
    for k in (1, 2, 3):
        dst = (me + k) % N_DEV
        chunks_ref[k, :, :] = build_chunk(dst)
        d = pltpu.make_async_remote_copy(
            src_ref=chunks_ref.at[k],
            dst_ref=recv_ref.at[k],
            send_sem=dsend.at[k],
            recv_sem=drecv.at[k],
            device_id=(dst,),
            device_id_type=pl.DeviceIdType.MESH,
        )
        d.start()
        drdmas.append(d)

    recv_ref[0, :, :] = build_chunk(me)

    for c in crdmas:
        c.wait()
    for d in drdmas:
        d.wait()


def _compact_body(recv_ref, idx_ref, mask_ref, out_ref, acc_ref):
    def outer(rb, _):
        base = rb * KB
        idxb = idx_ref[pl.ds(base, KB), :]
        acc_ref[...] = jnp.zeros((KB, D), jnp.bfloat16)

        def inner(j, _):
            @pl.when(mask_ref[rb, j] == 1)
            def _():
                cols = (jax.lax.broadcasted_iota(jnp.int32, (KB, KB), 1)
                        + j * KB)
                g = (cols == idxb).astype(jnp.bfloat16)
                rblk = recv_ref[pl.ds(j * KB, KB), :]
                acc_ref[...] += jax.lax.dot_general(
                    g, rblk, (((1,), (0,)), ((), ())),
                    preferred_element_type=jnp.float32,
                ).astype(jnp.bfloat16)
            return 0

        lax.fori_loop(0, NJ, inner, 0)
        out_ref[pl.ds(base, KB), :] = acc_ref[...]
        return 0

    lax.fori_loop(0, NR, outer, 0)


def kernel(x, dest):
    onehot = (dest[:, None] == jnp.arange(N_DEV)[None, :]).astype(jnp.int32)
    counts = onehot.sum(axis=0).astype(jnp.int32)
    ranks = (jnp.cumsum(onehot, axis=0) * onehot).sum(axis=1) - 1
    q = (dest.astype(jnp.int32) * MR + ranks).astype(jnp.int32)
    x_bf = x.astype(jnp.bfloat16)
    cnt_payload = jnp.zeros((1, 128), jnp.int32).at[0, :N_DEV].set(counts)

    recv, cnt_recv = pl.pallas_call(
        _a2a_body,
        out_shape=[
            jax.ShapeDtypeStruct((N_DEV, MR, D), jnp.bfloat16),
            jax.ShapeDtypeStruct((N_DEV, 128), jnp.int32),
        ],
        in_specs=[
            pl.BlockSpec(memory_space=pltpu.VMEM),
            pl.BlockSpec(memory_space=pltpu.VMEM),
            pl.BlockSpec(memory_space=pltpu.VMEM),
        ],
        out_specs=[
            pl.BlockSpec(memory_space=pltpu.VMEM),
            pl.BlockSpec(memory_space=pltpu.VMEM),
        ],
        scratch_shapes=[
            pltpu.VMEM((N_DEV, MR, D), jnp.bfloat16),
            pltpu.SemaphoreType.DMA((N_DEV,)),
            pltpu.SemaphoreType.DMA((N_DEV,)),
            pltpu.SemaphoreType.DMA((N_DEV,)),
            pltpu.SemaphoreType.DMA((N_DEV,)),
        ],
        compiler_params=pltpu.CompilerParams(collective_id=0),
    )(x_bf, q.reshape(T // KB1, 1, KB1), cnt_payload)

    me = lax.axis_index("i")
    c_slot = jnp.take(cnt_recv[:, :N_DEV], me, axis=1)
    slot_of_src = (me - jnp.arange(N_DEV)) % N_DEV
    c_src = c_slot[slot_of_src]
    cum = jnp.cumsum(c_src)
    starts = cum - c_src
    r = jnp.arange(T)
    p_of_r = (r[:, None] >= cum[None, :]).astype(jnp.int32).sum(axis=1)
    idx = (slot_of_src[p_of_r] * MR + (r - starts[p_of_r])).astype(jnp.int32)
    jb = idx.reshape(NR, KB) // KB
    mask = (jb[:, :, None] == jnp.arange(NJ)[None, None, :]).any(axis=1)

    out = pl.pallas_call(
        _compact_body,
        out_shape=jax.ShapeDtypeStruct((T, D), jnp.bfloat16),
        in_specs=[
            pl.BlockSpec(memory_space=pltpu.VMEM),
            pl.BlockSpec(memory_space=pltpu.VMEM),
            pl.BlockSpec(memory_space=pltpu.SMEM),
        ],
        out_specs=pl.BlockSpec(memory_space=pltpu.VMEM),
        scratch_shapes=[
            pltpu.VMEM((KB, D), jnp.bfloat16),
        ],
    )(recv.reshape(N_DEV * MR, D), idx.reshape(T, 1),
      mask.astype(jnp.int32))
    return out
